# baseline (speedup 1.0000x reference)
import functools

import jax
import jax.numpy as jnp
from jax.experimental import pallas as pl
from jax.experimental.pallas import tpu as pltpu


def _first_argmax(z, axis):
    zmax = jnp.max(z, axis=axis, keepdims=True)
    iota = jax.lax.broadcasted_iota(jnp.int32, z.shape, axis)
    big = jnp.int32(z.shape[axis])
    return jnp.min(jnp.where(z >= zmax, iota, big), axis=axis, keepdims=True)


def _fused_body(cat_ref, gam_ref, bet_ref, dep_ref, ua_ref, ud_ref, x_ref,
                o_ref, s_scr, t_scr, *, k):
    @pl.when(pl.program_id(0) == 0)
    def _routing():
        g = -jnp.log(-jnp.log(ua_ref[...]))
        z = cat_ref[...][None, :, :] + g
        idx = _first_argmax(z, axis=2)
        iota = jax.lax.broadcasted_iota(jnp.int32, z.shape, 2)
        oh = (iota == idx).astype(jnp.float32)
        s = jnp.sum(oh * gam_ref[...][None, :, :], axis=2)
        t = jnp.sum(oh * bet_ref[...][None, :, :], axis=2)

        gd = -jnp.log(-jnp.log(ud_ref[...]))
        zd = dep_ref[...] + gd
        m = _first_argmax(zd, axis=1)

        A = jnp.ones_like(m, dtype=jnp.float32)
        C = jnp.zeros_like(A)
        S = jnp.where(m == 0, A, 0.0)
        T = jnp.zeros_like(A)
        for i in range(k):
            si = s[:, i:i + 1]
            ti = t[:, i:i + 1]
            A = si * A
            C = si * C + ti
            S = jnp.where(m == i + 1, A, S)
            T = jnp.where(m == i + 1, C, T)
        s_scr[...] = S
        t_scr[...] = T

    o_ref[...] = s_scr[...] * x_ref[...] + t_scr[...]


def kernel(input, cat_logits, gammas, betas_aug, depth_logits):
    B = input.shape[0]
    k, d = cat_logits.shape
    n_flat = 1
    for dim in input.shape[1:]:
        n_flat *= dim
    x = input.reshape(B, n_flat)

    key = jax.random.key(42)
    k_aug, k_depth = jax.random.split(key)
    ua = jax.random.uniform(k_aug, (B, k, d), minval=1e-6, maxval=1.0 - 1e-6)
    ud = jax.random.uniform(k_depth, (B, k + 1), minval=1e-6, maxval=1.0 - 1e-6)

    W = 3072
    grid = (pl.cdiv(n_flat, W),)
    out = pl.pallas_call(
        functools.partial(_fused_body, k=k),
        grid=grid,
        in_specs=[
            pl.BlockSpec((k, d), lambda j: (0, 0)),
            pl.BlockSpec((k, d), lambda j: (0, 0)),
            pl.BlockSpec((k, d), lambda j: (0, 0)),
            pl.BlockSpec((1, k + 1), lambda j: (0, 0)),
            pl.BlockSpec((B, k, d), lambda j: (0, 0, 0)),
            pl.BlockSpec((B, k + 1), lambda j: (0, 0)),
            pl.BlockSpec((B, W), lambda j: (0, j)),
        ],
        out_specs=pl.BlockSpec((B, W), lambda j: (0, j)),
        out_shape=jax.ShapeDtypeStruct((B, n_flat), jnp.float32),
        scratch_shapes=[
            pltpu.VMEM((B, 1), jnp.float32),
            pltpu.VMEM((B, 1), jnp.float32),
        ],
    )(cat_logits, gammas, betas_aug, depth_logits.reshape(1, k + 1), ua, ud, x)
    return out.reshape(input.shape)


# speedup vs baseline: 1.0426x; 1.0426x over previous
import functools

import jax
import jax.numpy as jnp
from jax.experimental import pallas as pl
from jax.experimental.pallas import tpu as pltpu


def _first_argmax(z, axis):
    zmax = jnp.max(z, axis=axis, keepdims=True)
    iota = jax.lax.broadcasted_iota(jnp.int32, z.shape, axis)
    big = jnp.int32(z.shape[axis])
    return jnp.min(jnp.where(z >= zmax, iota, big), axis=axis, keepdims=True)


def _fused_body(cat_ref, gam_ref, bet_ref, dep_ref, ua_ref, ud_ref, x_ref,
                o_ref, s_scr, t_scr, *, k, q):
    @pl.when(pl.program_id(0) == 0)
    def _routing():
        g = -jnp.log(-jnp.log(ua_ref[...]))
        z = cat_ref[...][None, :, :] + g
        idx = _first_argmax(z, axis=2)
        iota = jax.lax.broadcasted_iota(jnp.int32, z.shape, 2)
        oh = (iota == idx).astype(jnp.float32)
        s = jnp.sum(oh * gam_ref[...][None, :, :], axis=2)
        t = jnp.sum(oh * bet_ref[...][None, :, :], axis=2)

        gd = -jnp.log(-jnp.log(ud_ref[...]))
        zd = dep_ref[...] + gd
        m = _first_argmax(zd, axis=1)

        A = jnp.ones_like(m, dtype=jnp.float32)
        C = jnp.zeros_like(A)
        S = jnp.where(m == 0, A, 0.0)
        T = jnp.zeros_like(A)
        for i in range(k):
            si = s[:, i:i + 1]
            ti = t[:, i:i + 1]
            A = si * A
            C = si * C + ti
            S = jnp.where(m == i + 1, A, S)
            T = jnp.where(m == i + 1, C, T)
        s_scr[...] = S
        t_scr[...] = T

    j = pl.program_id(0)
    s = s_scr[pl.ds(j * q, q), :].reshape(q, 1, 1)
    t = t_scr[pl.ds(j * q, q), :].reshape(q, 1, 1)
    o_ref[...] = s * x_ref[...] + t


def kernel(input, cat_logits, gammas, betas_aug, depth_logits):
    B = input.shape[0]
    k, d = cat_logits.shape
    n_flat = 1
    for dim in input.shape[1:]:
        n_flat *= dim
    rows = n_flat // 128
    x = input.reshape(B, rows, 128)

    key = jax.random.key(42)
    k_aug, k_depth = jax.random.split(key)
    ua = jax.random.uniform(k_aug, (B, k, d), minval=1e-6, maxval=1.0 - 1e-6)
    ud = jax.random.uniform(k_depth, (B, k + 1), minval=1e-6, maxval=1.0 - 1e-6)

    q = 4
    grid = (B // q,)
    out = pl.pallas_call(
        functools.partial(_fused_body, k=k, q=q),
        grid=grid,
        in_specs=[
            pl.BlockSpec((k, d), lambda j: (0, 0)),
            pl.BlockSpec((k, d), lambda j: (0, 0)),
            pl.BlockSpec((k, d), lambda j: (0, 0)),
            pl.BlockSpec((1, k + 1), lambda j: (0, 0)),
            pl.BlockSpec((B, k, d), lambda j: (0, 0, 0)),
            pl.BlockSpec((B, k + 1), lambda j: (0, 0)),
            pl.BlockSpec((q, rows, 128), lambda j: (j, 0, 0)),
        ],
        out_specs=pl.BlockSpec((q, rows, 128), lambda j: (j, 0, 0)),
        out_shape=jax.ShapeDtypeStruct((B, rows, 128), jnp.float32),
        scratch_shapes=[
            pltpu.VMEM((B, 1), jnp.float32),
            pltpu.VMEM((B, 1), jnp.float32),
        ],
    )(cat_logits, gammas, betas_aug, depth_logits.reshape(1, k + 1), ua, ud, x)
    return out.reshape(input.shape)
